# baseline (speedup 1.0000x reference)
import jax
import jax.numpy as jnp
from jax import lax
from jax.experimental import pallas as pl
from jax.experimental.pallas import tpu as pltpu, tpu_sc as plsc

N_NODES = 10000
N_EDGES = 160000
D = 256
DH = 128
DQ = 64
NS = 16
E_CHUNK = 40
CW = 50
W = 5
DEPTH = 4
EDGES_PER_TILE = N_EDGES // NS
EDGES_PER_WAVE = CW * E_CHUNK
ROWS_PER_TILE = 624
ROWS_TAIL = N_NODES - NS * ROWS_PER_TILE


def _body(feat_hbm, src3_hbm, dst4_hbm, efeat_hbm, out_hbm,
          acc, feat_sp, src_w, dst_w, fbuf, ebuf, gsem, esem, ssem):
    c = lax.axis_index("c")
    s = lax.axis_index("s")
    r0 = s * ROWS_PER_TILE
    t0 = NS * ROWS_PER_TILE

    def compute(p):
        fb = fbuf.at[p]
        eb = ebuf.at[p]
        def row(r, rc):
            for j in range(DQ // 16):
                sl = pl.ds(j * 16, 16)
                eb[r, sl] = jnp.maximum(fb[r, sl] + eb[r, sl], 0.0)
            return rc
        lax.fori_loop(0, E_CHUNK, row, 0)

    def run_pass(k, kcarry):
        col0 = pl.multiple_of(c * DH + k * DQ, DQ)

        pltpu.sync_copy(feat_hbm.at[pl.ds(r0, ROWS_PER_TILE),
                                    pl.ds(col0, DQ)],
                        acc.at[pl.ds(r0, ROWS_PER_TILE)])
        pltpu.sync_copy(feat_hbm.at[pl.ds(r0, ROWS_PER_TILE),
                                    pl.ds(col0, DQ)],
                        feat_sp.at[pl.ds(r0, ROWS_PER_TILE)])
        @pl.when(s == NS - 1)
        def _():
            pltpu.sync_copy(feat_hbm.at[pl.ds(t0, ROWS_TAIL),
                                        pl.ds(col0, DQ)],
                            acc.at[pl.ds(t0, ROWS_TAIL)])
            pltpu.sync_copy(feat_hbm.at[pl.ds(t0, ROWS_TAIL),
                                        pl.ds(col0, DQ)],
                            feat_sp.at[pl.ds(t0, ROWS_TAIL)])
        plsc.subcore_barrier()

        def wave(w, carry):
            pltpu.sync_copy(src3_hbm.at[s, w], src_w)
            pltpu.sync_copy(dst4_hbm.at[s, w], dst_w)

            gd = [None] * CW
            ed = [None] * CW
            sd = [None] * CW

            def issue(i):
                p = i % DEPTH
                base = pl.multiple_of(
                    s * EDGES_PER_TILE + w * EDGES_PER_WAVE + i * E_CHUNK, 8)
                gd[i] = pltpu.async_copy(
                    feat_sp.at[src_w.at[pl.ds(i * E_CHUNK, E_CHUNK)]],
                    fbuf.at[p], gsem.at[p])
                ed[i] = pltpu.async_copy(
                    efeat_hbm.at[pl.ds(base, E_CHUNK), pl.ds(col0, DQ)],
                    ebuf.at[p], esem.at[p])

            for j in range(DEPTH - 1):
                issue(j)
            for i in range(CW):
                p = i % DEPTH
                ni = i + DEPTH - 1
                if ni < CW:
                    if i >= 1:
                        sd[i - 1].wait()
                    issue(ni)
                gd[i].wait()
                ed[i].wait()
                compute(p)
                sd[i] = pltpu.async_copy(
                    ebuf.at[p], acc.at[dst_w.at[i]], ssem.at[p], add=True)
            for i in range(CW - DEPTH, CW):
                sd[i].wait()
            return carry

        lax.fori_loop(0, W, wave, 0)

        plsc.subcore_barrier()
        pltpu.sync_copy(acc.at[pl.ds(r0, ROWS_PER_TILE)],
                        out_hbm.at[pl.ds(r0, ROWS_PER_TILE),
                                   pl.ds(col0, DQ)])
        @pl.when(s == NS - 1)
        def _():
            pltpu.sync_copy(acc.at[pl.ds(t0, ROWS_TAIL)],
                            out_hbm.at[pl.ds(t0, ROWS_TAIL),
                                       pl.ds(col0, DQ)])
        plsc.subcore_barrier()
        return kcarry

    lax.fori_loop(0, 2, run_pass, 0)


def kernel(feat, edge_index, efeat):
    src3 = edge_index[0].astype(jnp.int32).reshape(NS, W, EDGES_PER_WAVE)
    dst4 = edge_index[1].astype(jnp.int32).reshape(NS, W, CW, E_CHUNK)

    run = pl.kernel(
        _body,
        out_type=jax.ShapeDtypeStruct((N_NODES, D), jnp.float32),
        mesh=plsc.VectorSubcoreMesh(core_axis_name="c", subcore_axis_name="s"),
        compiler_params=pltpu.CompilerParams(use_tc_tiling_on_sc=False),
        scratch_types=[
            pltpu.VMEM_SHARED((N_NODES, DQ), jnp.float32),
            pltpu.VMEM_SHARED((N_NODES, DQ), jnp.float32),
            pltpu.VMEM((EDGES_PER_WAVE,), jnp.int32),
            pltpu.VMEM((CW, E_CHUNK), jnp.int32),
            pltpu.VMEM((DEPTH, E_CHUNK, DQ), jnp.float32),
            pltpu.VMEM((DEPTH, E_CHUNK, DQ), jnp.float32),
            pltpu.SemaphoreType.DMA((DEPTH,)),
            pltpu.SemaphoreType.DMA((DEPTH,)),
            pltpu.SemaphoreType.DMA((DEPTH,)),
        ],
    )
    return run(feat, src3, dst4, efeat)


# speedup vs baseline: 1.1047x; 1.1047x over previous
import jax
import jax.numpy as jnp
from jax import lax
from jax.experimental import pallas as pl
from jax.experimental.pallas import tpu as pltpu, tpu_sc as plsc

N_NODES = 10000
N_EDGES = 160000
D = 256
DH = 128
NS = 16
E_CHUNK = 40
EDGES_PER_TILE = N_EDGES // NS
CHUNKS = EDGES_PER_TILE // E_CHUNK
GROUPS = CHUNKS // 2
ROWS_PER_TILE = 624
ROWS_TAIL = N_NODES - NS * ROWS_PER_TILE


def _body(fcat_hbm, src2_hbm, dst3_hbm, efeat_hbm, out_hbm,
          acc, src_v, dst_v, fbuf, ebuf, gsem, esem, ssem):
    c = lax.axis_index("c")
    s = lax.axis_index("s")
    col0 = pl.multiple_of(c * DH, DH)
    r0 = s * ROWS_PER_TILE
    t0 = NS * ROWS_PER_TILE
    feat_view = fcat_hbm.at[c]

    pltpu.sync_copy(src2_hbm.at[s], src_v)
    pltpu.sync_copy(dst3_hbm.at[s], dst_v)

    pltpu.sync_copy(feat_view.at[pl.ds(r0, ROWS_PER_TILE)],
                    acc.at[pl.ds(r0, ROWS_PER_TILE)])
    @pl.when(s == NS - 1)
    def _():
        pltpu.sync_copy(feat_view.at[pl.ds(t0, ROWS_TAIL)],
                        acc.at[pl.ds(t0, ROWS_TAIL)])
    plsc.subcore_barrier()

    def issue_loads(i, p):
        base = pl.multiple_of(s * EDGES_PER_TILE + i * E_CHUNK, 8)
        g = pltpu.async_copy(
            feat_view.at[src_v.at[pl.ds(i * E_CHUNK, E_CHUNK)]],
            fbuf.at[p], gsem.at[p])
        e = pltpu.async_copy(
            efeat_hbm.at[pl.ds(base, E_CHUNK), pl.ds(col0, DH)],
            ebuf.at[p], esem.at[p])
        return g, e

    def wait_loads(i, p):
        pltpu.make_async_copy(
            feat_view.at[src_v.at[pl.ds(i * E_CHUNK, E_CHUNK)]],
            fbuf.at[p], gsem.at[p]).wait()
        pltpu.make_async_copy(
            efeat_hbm.at[pl.ds(i * E_CHUNK, E_CHUNK), pl.ds(col0, DH)],
            ebuf.at[p], esem.at[p]).wait()

    def wait_scatter(i, p):
        pltpu.make_async_copy(
            ebuf.at[p], acc.at[dst_v.at[i]], ssem.at[p]).wait()

    def scatter(i, p):
        return pltpu.async_copy(
            ebuf.at[p], acc.at[dst_v.at[i]], ssem.at[p], add=True)

    def compute(p):
        fb = fbuf.at[p]
        eb = ebuf.at[p]
        def row(r, rc):
            for j in range(DH // 16):
                sl = pl.ds(j * 16, 16)
                eb[r, sl] = jnp.maximum(fb[r, sl] + eb[r, sl], 0.0)
            return rc
        lax.fori_loop(0, E_CHUNK, row, 0)

    issue_loads(0, 0)

    def group(g, carry):
        a = 2 * g
        b = a + 1
        @pl.when(g > 0)
        def _():
            wait_scatter(a - 1, 1)
        issue_loads(b, 1)
        wait_loads(a, 0)
        compute(0)
        sa = scatter(a, 0)
        sa.wait()
        @pl.when(g < GROUPS - 1)
        def _():
            issue_loads(a + 2, 0)
        wait_loads(b, 1)
        compute(1)
        scatter(b, 1)
        return carry

    lax.fori_loop(0, GROUPS, group, 0)
    wait_scatter(CHUNKS - 1, 1)

    plsc.subcore_barrier()
    pltpu.sync_copy(acc.at[pl.ds(r0, ROWS_PER_TILE)],
                    out_hbm.at[pl.ds(r0, ROWS_PER_TILE), pl.ds(col0, DH)])
    @pl.when(s == NS - 1)
    def _():
        pltpu.sync_copy(acc.at[pl.ds(t0, ROWS_TAIL)],
                        out_hbm.at[pl.ds(t0, ROWS_TAIL), pl.ds(col0, DH)])


def kernel(feat, edge_index, efeat):
    src2 = edge_index[0].astype(jnp.int32).reshape(NS, EDGES_PER_TILE)
    dst3 = edge_index[1].astype(jnp.int32).reshape(NS, CHUNKS, E_CHUNK)
    fcat = jnp.stack([feat[:, :DH], feat[:, DH:]])

    run = pl.kernel(
        _body,
        out_type=jax.ShapeDtypeStruct((N_NODES, D), jnp.float32),
        mesh=plsc.VectorSubcoreMesh(core_axis_name="c", subcore_axis_name="s"),
        compiler_params=pltpu.CompilerParams(use_tc_tiling_on_sc=False),
        scratch_types=[
            pltpu.VMEM_SHARED((N_NODES, DH), jnp.float32),
            pltpu.VMEM((EDGES_PER_TILE,), jnp.int32),
            pltpu.VMEM((CHUNKS, E_CHUNK), jnp.int32),
            pltpu.VMEM((2, E_CHUNK, DH), jnp.float32),
            pltpu.VMEM((2, E_CHUNK, DH), jnp.float32),
            pltpu.SemaphoreType.DMA((2,)),
            pltpu.SemaphoreType.DMA((2,)),
            pltpu.SemaphoreType.DMA((2,)),
        ],
    )
    return run(fcat, src2, dst3, efeat)


# speedup vs baseline: 1.8511x; 1.6756x over previous
import jax
import jax.numpy as jnp
from jax import lax
from jax.experimental import pallas as pl
from jax.experimental.pallas import tpu as pltpu, tpu_sc as plsc

N_NODES = 10000
N_EDGES = 160000
D = 256
DH = 128
NS = 16
E_CHUNK = 40
CW = 50
W = 5
DEPTH = 4
EDGES_PER_TILE = N_EDGES // NS
EDGES_PER_WAVE = CW * E_CHUNK
ROWS_PER_TILE = 624
ROWS_TAIL = N_NODES - NS * ROWS_PER_TILE


def _body(feat_hbm, src3_hbm, dst4_hbm, efeat_hbm, out_hbm,
          acc, src_w, dst_w, fbuf, ebuf, gsem, esem, ssem):
    c = lax.axis_index("c")
    s = lax.axis_index("s")
    col0 = pl.multiple_of(c * DH, DH)

    r0 = s * ROWS_PER_TILE
    pltpu.sync_copy(feat_hbm.at[pl.ds(r0, ROWS_PER_TILE), pl.ds(col0, DH)],
                    acc.at[pl.ds(r0, ROWS_PER_TILE)])
    @pl.when(s == NS - 1)
    def _():
        t0 = NS * ROWS_PER_TILE
        pltpu.sync_copy(feat_hbm.at[pl.ds(t0, ROWS_TAIL), pl.ds(col0, DH)],
                        acc.at[pl.ds(t0, ROWS_TAIL)])
    plsc.subcore_barrier()

    def compute(p):
        fb = fbuf.at[p]
        eb = ebuf.at[p]
        def row(r, rc):
            for j in range(DH // 16):
                sl = pl.ds(j * 16, 16)
                eb[r, sl] = jnp.maximum(fb[r, sl] + eb[r, sl], 0.0)
            return rc
        lax.fori_loop(0, E_CHUNK, row, 0)

    def wave(w, carry):
        pltpu.sync_copy(src3_hbm.at[s, w], src_w)
        pltpu.sync_copy(dst4_hbm.at[s, w], dst_w)

        gd = [None] * CW
        ed = [None] * CW
        sd = [None] * CW

        def issue_g(i):
            p = i % DEPTH
            gd[i] = pltpu.async_copy(
                feat_hbm.at[src_w.at[pl.ds(i * E_CHUNK, E_CHUNK)],
                            pl.ds(col0, DH)],
                fbuf.at[p], gsem.at[p])

        def issue_e(i):
            p = i % DEPTH
            base = pl.multiple_of(
                s * EDGES_PER_TILE + w * EDGES_PER_WAVE + i * E_CHUNK, 8)
            ed[i] = pltpu.async_copy(
                efeat_hbm.at[pl.ds(base, E_CHUNK), pl.ds(col0, DH)],
                ebuf.at[p], esem.at[p])

        for j in range(DEPTH - 1):
            issue_g(j)
            issue_e(j)
        for i in range(CW):
            p = i % DEPTH
            ni = i + DEPTH - 1
            if ni < CW:
                issue_g(ni)
                if i >= 1:
                    sd[i - 1].wait()
                issue_e(ni)
            gd[i].wait()
            ed[i].wait()
            compute(p)
            sd[i] = pltpu.async_copy(
                ebuf.at[p], acc.at[dst_w.at[i]], ssem.at[p], add=True)
        for i in range(CW - DEPTH, CW):
            sd[i].wait()
        return carry

    lax.fori_loop(0, W, wave, 0)

    plsc.subcore_barrier()
    pltpu.sync_copy(acc.at[pl.ds(r0, ROWS_PER_TILE)],
                    out_hbm.at[pl.ds(r0, ROWS_PER_TILE), pl.ds(col0, DH)])
    @pl.when(s == NS - 1)
    def _():
        t0 = NS * ROWS_PER_TILE
        pltpu.sync_copy(acc.at[pl.ds(t0, ROWS_TAIL)],
                        out_hbm.at[pl.ds(t0, ROWS_TAIL), pl.ds(col0, DH)])


def kernel(feat, edge_index, efeat):
    src3 = edge_index[0].astype(jnp.int32).reshape(NS, W, EDGES_PER_WAVE)
    dst4 = edge_index[1].astype(jnp.int32).reshape(NS, W, CW, E_CHUNK)

    run = pl.kernel(
        _body,
        out_type=jax.ShapeDtypeStruct((N_NODES, D), jnp.float32),
        mesh=plsc.VectorSubcoreMesh(core_axis_name="c", subcore_axis_name="s"),
        scratch_types=[
            pltpu.VMEM_SHARED((N_NODES, DH), jnp.float32),
            pltpu.VMEM((EDGES_PER_WAVE,), jnp.int32),
            pltpu.VMEM((CW, E_CHUNK), jnp.int32),
            pltpu.VMEM((DEPTH, E_CHUNK, DH), jnp.float32),
            pltpu.VMEM((DEPTH, E_CHUNK, DH), jnp.float32),
            pltpu.SemaphoreType.DMA((DEPTH,)),
            pltpu.SemaphoreType.DMA((DEPTH,)),
            pltpu.SemaphoreType.DMA((DEPTH,)),
        ],
    )
    return run(feat, src3, dst4, efeat)


# speedup vs baseline: 1.8845x; 1.0181x over previous
import jax
import jax.numpy as jnp
from jax import lax
from jax.experimental import pallas as pl
from jax.experimental.pallas import tpu as pltpu, tpu_sc as plsc

N_NODES = 10000
N_EDGES = 160000
D = 256
DH = 128
NS = 16
E_CHUNK = 80
CW = 25
W = 5
DEPTH = 2
EDGES_PER_TILE = N_EDGES // NS
EDGES_PER_WAVE = CW * E_CHUNK
ROWS_PER_TILE = 624
ROWS_TAIL = N_NODES - NS * ROWS_PER_TILE


def _body(feat_hbm, src3_hbm, dst4_hbm, efeat_hbm, out_hbm,
          acc, src_w, dst_w, fbuf, ebuf, gsem, esem, ssem):
    c = lax.axis_index("c")
    s = lax.axis_index("s")
    col0 = pl.multiple_of(c * DH, DH)

    r0 = s * ROWS_PER_TILE
    pltpu.sync_copy(feat_hbm.at[pl.ds(r0, ROWS_PER_TILE), pl.ds(col0, DH)],
                    acc.at[pl.ds(r0, ROWS_PER_TILE)])
    @pl.when(s == NS - 1)
    def _():
        t0 = NS * ROWS_PER_TILE
        pltpu.sync_copy(feat_hbm.at[pl.ds(t0, ROWS_TAIL), pl.ds(col0, DH)],
                        acc.at[pl.ds(t0, ROWS_TAIL)])
    plsc.subcore_barrier()

    def compute(p):
        fb = fbuf.at[p]
        eb = ebuf.at[p]
        def row(r, rc):
            for j in range(DH // 16):
                sl = pl.ds(j * 16, 16)
                eb[r, sl] = jnp.maximum(fb[r, sl] + eb[r, sl], 0.0)
            return rc
        lax.fori_loop(0, E_CHUNK, row, 0)

    def wave(w, carry):
        pltpu.sync_copy(src3_hbm.at[s, w], src_w)
        pltpu.sync_copy(dst4_hbm.at[s, w], dst_w)

        gd = [None] * CW
        ed = [None] * CW
        sd = [None] * CW

        def issue_g(i):
            p = i % DEPTH
            gd[i] = pltpu.async_copy(
                feat_hbm.at[src_w.at[pl.ds(i * E_CHUNK, E_CHUNK)],
                            pl.ds(col0, DH)],
                fbuf.at[p], gsem.at[p])

        def issue_e(i):
            p = i % DEPTH
            base = pl.multiple_of(
                s * EDGES_PER_TILE + w * EDGES_PER_WAVE + i * E_CHUNK, 8)
            ed[i] = pltpu.async_copy(
                efeat_hbm.at[pl.ds(base, E_CHUNK), pl.ds(col0, DH)],
                ebuf.at[p], esem.at[p])

        for j in range(DEPTH - 1):
            issue_g(j)
            issue_e(j)
        for i in range(CW):
            p = i % DEPTH
            ni = i + DEPTH - 1
            if ni < CW:
                issue_g(ni)
                if i >= 1:
                    sd[i - 1].wait()
                issue_e(ni)
            gd[i].wait()
            ed[i].wait()
            compute(p)
            sd[i] = pltpu.async_copy(
                ebuf.at[p], acc.at[dst_w.at[i]], ssem.at[p], add=True)
        for i in range(CW - DEPTH, CW):
            sd[i].wait()
        return carry

    lax.fori_loop(0, W, wave, 0)

    plsc.subcore_barrier()
    pltpu.sync_copy(acc.at[pl.ds(r0, ROWS_PER_TILE)],
                    out_hbm.at[pl.ds(r0, ROWS_PER_TILE), pl.ds(col0, DH)])
    @pl.when(s == NS - 1)
    def _():
        t0 = NS * ROWS_PER_TILE
        pltpu.sync_copy(acc.at[pl.ds(t0, ROWS_TAIL)],
                        out_hbm.at[pl.ds(t0, ROWS_TAIL), pl.ds(col0, DH)])


def kernel(feat, edge_index, efeat):
    src3 = edge_index[0].astype(jnp.int32).reshape(NS, W, EDGES_PER_WAVE)
    dst4 = edge_index[1].astype(jnp.int32).reshape(NS, W, CW, E_CHUNK)

    run = pl.kernel(
        _body,
        out_type=jax.ShapeDtypeStruct((N_NODES, D), jnp.float32),
        mesh=plsc.VectorSubcoreMesh(core_axis_name="c", subcore_axis_name="s"),
        scratch_types=[
            pltpu.VMEM_SHARED((N_NODES, DH), jnp.float32),
            pltpu.VMEM((EDGES_PER_WAVE,), jnp.int32),
            pltpu.VMEM((CW, E_CHUNK), jnp.int32),
            pltpu.VMEM((DEPTH, E_CHUNK, DH), jnp.float32),
            pltpu.VMEM((DEPTH, E_CHUNK, DH), jnp.float32),
            pltpu.SemaphoreType.DMA((DEPTH,)),
            pltpu.SemaphoreType.DMA((DEPTH,)),
            pltpu.SemaphoreType.DMA((DEPTH,)),
        ],
    )
    return run(feat, src3, dst4, efeat)
